# Initial kernel scaffold; baseline (speedup 1.0000x reference)
import jax
import jax.numpy as jnp
from jax.experimental import pallas as pl


def kernel(triplet_idx, entity_embedding, relation_embedding):
    raise NotImplementedError("write your pallas kernel here")


# speedup vs baseline: 1.3781x; 1.3781x over previous
import functools

import jax
import jax.numpy as jnp
from jax import lax
from jax.experimental import pallas as pl
from jax.experimental.pallas import tpu as pltpu
from jax.experimental.pallas import tpu_sc as plsc

B = 16384
D = 128
NC = 2
NS = 16
NW = NC * NS
B_PER_W = B // NW
CHUNK = 128
NCH = B_PER_W // CHUNK
LANES = 16
DG = D // LANES


def _body(hidx_hbm, ridx_hbm, tidx_hbm, ent_hbm, rel_hbm, out_hbm,
          hidx_v, ridx_v, tidx_v, hbuf, rbuf, tbuf, acc16, out_v, sem):
    wid = lax.axis_index("s") * NC + lax.axis_index("c")

    pltpu.sync_copy(hidx_hbm.at[wid], hidx_v)
    pltpu.sync_copy(ridx_hbm.at[wid], ridx_v)
    pltpu.sync_copy(tidx_hbm.at[wid], tidx_v)

    for j in range(NCH):
        cps = [
            pltpu.async_copy(ent_hbm.at[hidx_v.at[j]], hbuf, sem),
            pltpu.async_copy(rel_hbm.at[ridx_v.at[j]], rbuf, sem),
            pltpu.async_copy(ent_hbm.at[tidx_v.at[j]], tbuf, sem),
        ]
        for cp in cps:
            cp.wait()

        def group(g, _, j=j):
            base = g * LANES
            for i in range(LANES):
                row = base + i
                acc = (hbuf[row, pl.ds(0, LANES)]
                       * rbuf[row, pl.ds(0, LANES)]
                       * tbuf[row, pl.ds(0, LANES)])
                for dg in range(1, DG):
                    acc = acc + (hbuf[row, pl.ds(dg * LANES, LANES)]
                                 * rbuf[row, pl.ds(dg * LANES, LANES)]
                                 * tbuf[row, pl.ds(dg * LANES, LANES)])
                acc16[pl.ds(i * LANES, LANES)] = acc
            rows = lax.iota(jnp.int32, LANES) * LANES
            score = plsc.load_gather(acc16, [rows])
            for l in range(1, LANES):
                score = score + plsc.load_gather(acc16, [rows + l])
            out_v[pl.ds(j * CHUNK + base, LANES)] = score
            return 0

        lax.fori_loop(0, CHUNK // LANES, group, 0)

    pltpu.sync_copy(out_v, out_hbm.at[pl.ds(wid * B_PER_W, B_PER_W)])


@functools.partial(jax.jit, static_argnames=())
def _run(hidx, ridx, tidx, ent, rel):
    mesh = plsc.VectorSubcoreMesh(core_axis_name="c", subcore_axis_name="s")
    return pl.kernel(
        _body,
        out_type=jax.ShapeDtypeStruct((B,), jnp.float32),
        mesh=mesh,
        compiler_params=pltpu.CompilerParams(needs_layout_passes=False),
        scratch_types=[
            pltpu.VMEM((NCH, CHUNK), jnp.int32),
            pltpu.VMEM((NCH, CHUNK), jnp.int32),
            pltpu.VMEM((NCH, CHUNK), jnp.int32),
            pltpu.VMEM((CHUNK, D), jnp.float32),
            pltpu.VMEM((CHUNK, D), jnp.float32),
            pltpu.VMEM((CHUNK, D), jnp.float32),
            pltpu.VMEM((LANES * LANES,), jnp.float32),
            pltpu.VMEM((B_PER_W,), jnp.float32),
            pltpu.SemaphoreType.DMA,
        ],
    )(hidx, ridx, tidx, ent, rel)


def kernel(triplet_idx, entity_embedding, relation_embedding):
    idx = triplet_idx.astype(jnp.int32)
    hidx = idx[:, 0].reshape(NW, NCH, CHUNK)
    ridx = idx[:, 1].reshape(NW, NCH, CHUNK)
    tidx = idx[:, 2].reshape(NW, NCH, CHUNK)
    return _run(hidx, ridx, tidx, entity_embedding, relation_embedding)


# speedup vs baseline: 1.5904x; 1.1540x over previous
import functools

import jax
import jax.numpy as jnp
from jax import lax
from jax.experimental import pallas as pl
from jax.experimental.pallas import tpu as pltpu
from jax.experimental.pallas import tpu_sc as plsc

B = 16384
D = 128
NC = 2
NS = 16
NW = NC * NS
B_PER_W = B // NW
CHUNK = 128
NCH = B_PER_W // CHUNK
LANES = 16
DG = D // LANES


def _body(hidx_hbm, ridx_hbm, tidx_hbm, ent_hbm, rel_hbm, out_hbm,
          hidx_v, ridx_v, tidx_v,
          hbuf0, rbuf0, tbuf0, hbuf1, rbuf1, tbuf1,
          acc16, out_v, sem0, sem1):
    wid = lax.axis_index("s") * NC + lax.axis_index("c")

    pltpu.sync_copy(hidx_hbm.at[wid], hidx_v)
    pltpu.sync_copy(ridx_hbm.at[wid], ridx_v)
    pltpu.sync_copy(tidx_hbm.at[wid], tidx_v)

    bufs = [(hbuf0, rbuf0, tbuf0), (hbuf1, rbuf1, tbuf1)]
    sems = [sem0, sem1]

    def fire(j):
        h, r, t = bufs[j % 2]
        s = sems[j % 2]
        return [
            pltpu.async_copy(ent_hbm.at[hidx_v.at[j]], h, s),
            pltpu.async_copy(rel_hbm.at[ridx_v.at[j]], r, s),
            pltpu.async_copy(ent_hbm.at[tidx_v.at[j]], t, s),
        ]

    inflight = {0: fire(0)}
    for j in range(NCH):
        if j + 1 < NCH:
            inflight[j + 1] = fire(j + 1)
        for cp in inflight.pop(j):
            cp.wait()
        hbuf, rbuf, tbuf = bufs[j % 2]

        def group(g, _, j=j, hbuf=hbuf, rbuf=rbuf, tbuf=tbuf):
            base = g * LANES
            for i in range(LANES):
                row = base + i
                acc = (hbuf[row, pl.ds(0, LANES)]
                       * rbuf[row, pl.ds(0, LANES)]
                       * tbuf[row, pl.ds(0, LANES)])
                for dg in range(1, DG):
                    acc = acc + (hbuf[row, pl.ds(dg * LANES, LANES)]
                                 * rbuf[row, pl.ds(dg * LANES, LANES)]
                                 * tbuf[row, pl.ds(dg * LANES, LANES)])
                acc16[pl.ds(i * LANES, LANES)] = acc
            rows = lax.iota(jnp.int32, LANES) * LANES
            score = plsc.load_gather(acc16, [rows])
            for l in range(1, LANES):
                score = score + plsc.load_gather(acc16, [rows + l])
            out_v[pl.ds(j * CHUNK + base, LANES)] = score
            return 0

        lax.fori_loop(0, CHUNK // LANES, group, 0)

    pltpu.sync_copy(out_v, out_hbm.at[pl.ds(wid * B_PER_W, B_PER_W)])


@functools.partial(jax.jit, static_argnames=())
def _run(hidx, ridx, tidx, ent, rel):
    mesh = plsc.VectorSubcoreMesh(core_axis_name="c", subcore_axis_name="s")
    return pl.kernel(
        _body,
        out_type=jax.ShapeDtypeStruct((B,), jnp.float32),
        mesh=mesh,
        compiler_params=pltpu.CompilerParams(needs_layout_passes=False),
        scratch_types=[
            pltpu.VMEM((NCH, CHUNK), jnp.int32),
            pltpu.VMEM((NCH, CHUNK), jnp.int32),
            pltpu.VMEM((NCH, CHUNK), jnp.int32),
            pltpu.VMEM((CHUNK, D), jnp.float32),
            pltpu.VMEM((CHUNK, D), jnp.float32),
            pltpu.VMEM((CHUNK, D), jnp.float32),
            pltpu.VMEM((CHUNK, D), jnp.float32),
            pltpu.VMEM((CHUNK, D), jnp.float32),
            pltpu.VMEM((CHUNK, D), jnp.float32),
            pltpu.VMEM((LANES * LANES,), jnp.float32),
            pltpu.VMEM((B_PER_W,), jnp.float32),
            pltpu.SemaphoreType.DMA,
            pltpu.SemaphoreType.DMA,
        ],
    )(hidx, ridx, tidx, ent, rel)


def kernel(triplet_idx, entity_embedding, relation_embedding):
    idx = triplet_idx.astype(jnp.int32)
    hidx = idx[:, 0].reshape(NW, NCH, CHUNK)
    ridx = idx[:, 1].reshape(NW, NCH, CHUNK)
    tidx = idx[:, 2].reshape(NW, NCH, CHUNK)
    return _run(hidx, ridx, tidx, entity_embedding, relation_embedding)
